# Initial kernel scaffold; baseline (speedup 1.0000x reference)
import jax
import jax.numpy as jnp
from jax.experimental import pallas as pl


def kernel(input_seq, input_positions, emb_table, pos_table):
    raise NotImplementedError("write your pallas kernel here")


# speedup vs baseline: 1.6793x; 1.6793x over previous
import functools

import jax
import jax.numpy as jnp
from jax import lax
from jax.experimental import pallas as pl
from jax.experimental.pallas import tpu as pltpu
from jax.experimental.pallas import tpu_sc as plsc

N_VOCAB = 100000
N_POSITION = 4096
D_MODEL = 768
SEQ = 4096
BATCH = 4

NC = 2
NS = 16
NW = NC * NS

N_ROWS = SEQ * BATCH
RPW = N_ROWS // NW
CH = 64
NCHUNK = RPW // CH
LANES = 16
NVEC = D_MODEL // LANES


def _sc_body(seq_hbm, posidx_hbm, emb_hbm, pos_hbm, out_hbm,
             idx_t, idx_p, tok_v, pos_v, sem_t, sem_p):
    cid = lax.axis_index("c")
    sid = lax.axis_index("s")
    wid = sid * NC + cid

    pltpu.sync_copy(seq_hbm.at[wid], idx_t)
    pltpu.sync_copy(posidx_hbm.at[wid], idx_p)

    base = wid * RPW

    def chunk_body(c, carry):
        ct = pltpu.async_copy(emb_hbm.at[idx_t.at[c]], tok_v, sem_t)
        cp = pltpu.async_copy(pos_hbm.at[idx_p.at[c]], pos_v, sem_p)
        ct.wait()
        cp.wait()

        def row_body(r, carry2):
            for j in range(NVEC):
                sl = pl.ds(j * LANES, LANES)
                tok_v[r, sl] = tok_v[r, sl] + pos_v[r, sl]
            return carry2

        lax.fori_loop(0, CH, row_body, 0, unroll=False)

        off = pl.multiple_of(base + c * CH, CH)
        pltpu.sync_copy(tok_v, out_hbm.at[pl.ds(off, CH)])
        return carry

    lax.fori_loop(0, NCHUNK, chunk_body, 0, unroll=False)


@jax.jit
def kernel(input_seq, input_positions, emb_table, pos_table):
    seq_flat = input_seq.reshape(NW, NCHUNK, CH)
    pos_flat = input_positions.reshape(NW, NCHUNK, CH)

    mesh = plsc.VectorSubcoreMesh(core_axis_name="c", subcore_axis_name="s",
                                  num_cores=NC, num_subcores=NS)
    out = pl.kernel(
        _sc_body,
        out_type=jax.ShapeDtypeStruct((N_ROWS, D_MODEL), jnp.float32),
        mesh=mesh,
        scratch_types=[
            pltpu.VMEM((NCHUNK, CH), jnp.int32),
            pltpu.VMEM((NCHUNK, CH), jnp.int32),
            pltpu.VMEM((CH, D_MODEL), jnp.float32),
            pltpu.VMEM((CH, D_MODEL), jnp.float32),
            pltpu.SemaphoreType.DMA,
            pltpu.SemaphoreType.DMA,
        ],
    )(seq_flat, pos_flat, emb_table, pos_table)
    return out.reshape(SEQ, BATCH, D_MODEL)
